# baseline (speedup 1.0000x reference)
import functools

import jax
import jax.numpy as jnp
from jax import lax
from jax.experimental import pallas as pl
from jax.experimental.pallas import tpu as pltpu
from jax.experimental.pallas import tpu_sc as plsc

_LB = 128
_DP = 128


@functools.lru_cache(maxsize=None)
def _make_gather(B, H, V, D):
    n_slab = B * H // _LB
    info = plsc.get_sparse_core_info()
    NC, NS = info.num_cores, info.num_subcores
    NW = NC * NS
    per_w = n_slab // NW
    assert per_w * NW == n_slab and per_w % 2 == 0
    tb_per_h = B // _LB
    mesh = plsc.VectorSubcoreMesh(core_axis_name="c", subcore_axis_name="s")

    @functools.partial(
        pl.kernel,
        mesh=mesh,
        compiler_params=pltpu.CompilerParams(needs_layout_passes=False),
        out_type=jax.ShapeDtypeStruct((H, D, B), jnp.float32),
        scratch_types=[
            pltpu.VMEM((2, 1, _LB), jnp.int32),
            pltpu.VMEM((2, _LB, _DP), jnp.float32),
            pltpu.VMEM((2, D, _LB), jnp.float32),
            pltpu.SemaphoreType.DMA,
            pltpu.SemaphoreType.DMA,
            pltpu.SemaphoreType.DMA,
            pltpu.SemaphoreType.DMA,
            pltpu.SemaphoreType.DMA,
            pltpu.SemaphoreType.DMA,
        ],
    )
    def k(idx_hbm, table_hbm, out_hbm, idx_v, raw_v, tr_v,
          sg0, sg1, so0, so1, si0, si1):
        sg = (sg0, sg1)
        so = (so0, so1)
        si = (si0, si1)
        wid = lax.axis_index("s") * NC + lax.axis_index("c")
        s0 = wid * per_w

        def start_idx(s, b):
            pltpu.async_copy(idx_hbm.at[pl.ds(s0 + s, 1)], idx_v.at[b], si[b])

        def wait_idx(b):
            pltpu.make_async_copy(idx_hbm.at[pl.ds(0, 1)], idx_v.at[b], si[b]).wait()

        def start_gather(b):
            pltpu.async_copy(table_hbm.at[idx_v.at[b, 0]], raw_v.at[b], sg[b])

        def wait_gather(b):
            pltpu.make_async_copy(table_hbm.at[pl.ds(0, _LB)], raw_v.at[b], sg[b]).wait()

        def start_out(s, b):
            sa = s0 + s
            h = sa // tb_per_h
            tb = sa % tb_per_h
            pltpu.async_copy(tr_v.at[b], out_hbm.at[h, :, pl.ds(tb * _LB, _LB)], so[b])

        def wait_out(b):
            pltpu.make_async_copy(
                tr_v.at[b], out_hbm.at[0, :, pl.ds(0, _LB)], so[b]
            ).wait()

        def transpose(b):
            @plsc.parallel_loop(0, D * 8, 1, unroll=32)
            def dbody(i):
                d = i // 8
                bg = i % 8
                col = jnp.full((16,), d, dtype=jnp.int32)
                rows = lax.iota(jnp.int32, 16) + bg * 16
                v = plsc.load_gather(raw_v.at[b], [rows, col])
                tr_v[b, d, pl.ds(bg * 16, 16)] = v

        def pair(t, prefetch, first):
            g0 = 2 * t
            wait_gather(0)
            wait_idx(1)
            start_gather(1)
            if prefetch:
                start_idx(g0 + 2, 0)
            if not first:
                wait_out(0)
            transpose(0)
            start_out(g0, 0)
            wait_gather(1)
            if prefetch:
                wait_idx(0)
                start_gather(0)
                start_idx(g0 + 3, 1)
            if not first:
                wait_out(1)
            transpose(1)
            start_out(g0 + 1, 1)

        start_idx(0, 0)
        wait_idx(0)
        start_gather(0)
        start_idx(1, 1)
        T = per_w // 2
        pair(0, True, True)
        lax.fori_loop(1, T - 1, lambda t, c: (pair(t, True, False), c)[1], 0)
        pair(T - 1, False, False)
        wait_out(0)
        wait_out(1)

    return k


def kernel(input, table):
    B, H = input.shape
    V, D = table.shape
    idx2d = input.T.reshape(H * B // _LB, _LB).astype(jnp.int32)
    table_p = jnp.pad(table, ((0, 0), (0, _DP - D)))
    out = _make_gather(B, H, V, D)(idx2d, table_p)
    return jnp.transpose(out, (2, 0, 1))


# speedup vs baseline: 1.5720x; 1.5720x over previous
import functools

import jax
import jax.numpy as jnp
from jax import lax
from jax.experimental import pallas as pl
from jax.experimental.pallas import tpu as pltpu
from jax.experimental.pallas import tpu_sc as plsc

_CB = 128
_K = 2
_CH = _CB * _K
_DP = 128


@functools.lru_cache(maxsize=None)
def _make_gather(N, V, D):
    info = plsc.get_sparse_core_info()
    NC, NS = info.num_cores, info.num_subcores
    NW = NC * NS
    per_w = N // NW
    assert per_w * NW == N
    n_ch = per_w // _CH
    assert n_ch * _CH == per_w and n_ch % 2 == 0
    mesh = plsc.VectorSubcoreMesh(core_axis_name="c", subcore_axis_name="s")

    @functools.partial(
        pl.kernel,
        mesh=mesh,
        out_type=jax.ShapeDtypeStruct((N, _DP), jnp.float32),
        scratch_types=[
            pltpu.VMEM((2, _K, _CB), jnp.int32),
            pltpu.VMEM((2, _CH, _DP), jnp.float32),
            pltpu.SemaphoreType.DMA,
            pltpu.SemaphoreType.DMA,
            pltpu.SemaphoreType.DMA,
            pltpu.SemaphoreType.DMA,
            pltpu.SemaphoreType.DMA,
            pltpu.SemaphoreType.DMA,
        ],
    )
    def k(idx_hbm, table_hbm, out_hbm, idx_v, rows_v, sg0, sg1, so0, so1, si0, si1):
        sg = (sg0, sg1)
        so = (so0, so1)
        si = (si0, si1)
        wid = lax.axis_index("s") * NC + lax.axis_index("c")
        row0 = wid * (per_w // _CB)

        def start_idx(g, b):
            pltpu.async_copy(idx_hbm.at[pl.ds(row0 + g * _K, _K)], idx_v.at[b], si[b])

        def wait_idx(b):
            pltpu.make_async_copy(idx_hbm.at[pl.ds(0, _K)], idx_v.at[b], si[b]).wait()

        def start_gathers(b):
            for j in range(_K):
                pltpu.async_copy(
                    table_hbm.at[idx_v.at[b, j]],
                    rows_v.at[b, pl.ds(j * _CB, _CB)],
                    sg[b],
                )

        def wait_gathers(b):
            pltpu.make_async_copy(table_hbm.at[pl.ds(0, _CH)], rows_v.at[b], sg[b]).wait()

        def start_out(g, b):
            pltpu.async_copy(
                rows_v.at[b], out_hbm.at[pl.ds((row0 + g * _K) * _CB, _CH)], so[b]
            )

        def wait_out(b):
            pltpu.make_async_copy(rows_v.at[b], out_hbm.at[pl.ds(0, _CH)], so[b]).wait()

        def pair(t, prefetch):
            g0 = 2 * t
            wait_gathers(0)
            start_out(g0, 0)
            wait_idx(1)
            start_gathers(1)
            if prefetch:
                start_idx(g0 + 2, 0)
            wait_gathers(1)
            start_out(g0 + 1, 1)
            if prefetch:
                start_idx(g0 + 3, 1)
            wait_out(0)
            if prefetch:
                wait_idx(0)
                start_gathers(0)
            wait_out(1)

        start_idx(0, 0)
        wait_idx(0)
        start_gathers(0)
        start_idx(1, 1)
        lax.fori_loop(0, n_ch // 2 - 1, lambda t, c: (pair(t, True), c)[1], 0)
        pair(n_ch // 2 - 1, False)

    return k


def kernel(input, table):
    B, H = input.shape
    V, D = table.shape
    N = B * H
    idx2d = input.reshape(N // _CB, _CB).astype(jnp.int32)
    table_p = jnp.pad(table, ((0, 0), (0, _DP - D)))
    out = _make_gather(N, V, D)(idx2d, table_p)
    return out[:, :D].reshape(B, H, D)


# speedup vs baseline: 1.5844x; 1.0079x over previous
import functools

import jax
import jax.numpy as jnp
from jax import lax
from jax.experimental import pallas as pl
from jax.experimental.pallas import tpu as pltpu
from jax.experimental.pallas import tpu_sc as plsc

_CB = 128
_CH = _CB
_DP = 128
_NB = 4


@functools.lru_cache(maxsize=None)
def _make_gather(N, V, D):
    info = plsc.get_sparse_core_info()
    NC, NS = info.num_cores, info.num_subcores
    NW = NC * NS
    per_w = N // NW
    assert per_w * NW == N
    n_ch = per_w // _CH
    assert n_ch * _CH == per_w and n_ch % _NB == 0
    mesh = plsc.VectorSubcoreMesh(core_axis_name="c", subcore_axis_name="s")

    @functools.partial(
        pl.kernel,
        mesh=mesh,
        out_type=jax.ShapeDtypeStruct((N, _DP), jnp.float32),
        scratch_types=[
            pltpu.VMEM((_NB, 1, _CB), jnp.int32),
            pltpu.VMEM((_NB, _CH, _DP), jnp.float32),
            pltpu.SemaphoreType.DMA,
            pltpu.SemaphoreType.DMA,
            pltpu.SemaphoreType.DMA,
            pltpu.SemaphoreType.DMA,
            pltpu.SemaphoreType.DMA,
            pltpu.SemaphoreType.DMA,
            pltpu.SemaphoreType.DMA,
            pltpu.SemaphoreType.DMA,
            pltpu.SemaphoreType.DMA,
            pltpu.SemaphoreType.DMA,
            pltpu.SemaphoreType.DMA,
            pltpu.SemaphoreType.DMA,
        ],
    )
    def k(idx_hbm, table_hbm, out_hbm, idx_v, rows_v,
          sg0, sg1, sg2, sg3, so0, so1, so2, so3, si0, si1, si2, si3):
        sg = (sg0, sg1, sg2, sg3)
        so = (so0, so1, so2, so3)
        si = (si0, si1, si2, si3)
        wid = lax.axis_index("s") * NC + lax.axis_index("c")
        row0 = wid * n_ch

        def start_idx(g, b):
            pltpu.async_copy(idx_hbm.at[pl.ds(row0 + g, 1)], idx_v.at[b], si[b])

        def wait_idx(b):
            pltpu.make_async_copy(idx_hbm.at[pl.ds(0, 1)], idx_v.at[b], si[b]).wait()

        def start_gather(g, b):
            del g
            pltpu.async_copy(table_hbm.at[idx_v.at[b, 0]], rows_v.at[b], sg[b])

        def wait_gather(b):
            pltpu.make_async_copy(table_hbm.at[pl.ds(0, _CH)], rows_v.at[b], sg[b]).wait()

        def start_out(g, b):
            pltpu.async_copy(
                rows_v.at[b], out_hbm.at[pl.ds((row0 + g) * _CB, _CH)], so[b]
            )

        def wait_out(b):
            pltpu.make_async_copy(rows_v.at[b], out_hbm.at[pl.ds(0, _CH)], so[b]).wait()

        def sub_step(g, b, do_wait_out, start_g, start_i):
            bb = (b + 2) % _NB
            wait_gather(b)
            start_out(g, b)
            if do_wait_out:
                wait_out(bb)
            if start_g:
                wait_idx(bb)
                start_gather(g + 2, bb)
            if start_i:
                start_idx(g + 4, b)

        def quad(t, do_wait01, start_g01, start_i):
            g0 = _NB * t
            sub_step(g0 + 0, 0, do_wait01, start_g01, start_i)
            sub_step(g0 + 1, 1, do_wait01, start_g01, start_i)
            sub_step(g0 + 2, 2, True, start_i, start_i)
            sub_step(g0 + 3, 3, True, start_i, start_i)

        for b in range(_NB):
            start_idx(b, b)
        wait_idx(0)
        start_gather(0, 0)
        wait_idx(1)
        start_gather(1, 1)
        T = n_ch // _NB
        quad(0, False, True, True)
        lax.fori_loop(1, T - 1, lambda t, c: (quad(t, True, True, True), c)[1], 0)
        quad(T - 1, True, True, False)
        wait_out(2)
        wait_out(3)

    return k


def kernel(input, table):
    B, H = input.shape
    V, D = table.shape
    N = B * H
    idx2d = input.reshape(N // _CB, _CB).astype(jnp.int32)
    table_p = jnp.pad(table, ((0, 0), (0, _DP - D)))
    out = _make_gather(N, V, D)(idx2d, table_p)
    return out[:, :D].reshape(B, H, D)
